# baseline (speedup 1.0000x reference)
import functools

import jax
import jax.numpy as jnp
from jax import lax
from jax.experimental import pallas as pl
from jax.experimental.pallas import tpu as pltpu
from jax.experimental.pallas import tpu_sc as plsc

_NF = 26
_V = 100000
_E = 32
_B = 16384
_CHUNK = 4096
_NCHUNK = _B // _CHUNK
_UNROLL = 32


def _make_kernel():
  mesh = plsc.VectorSubcoreMesh(core_axis_name="c", subcore_axis_name="s")

  @functools.partial(
      pl.kernel,
      mesh=mesh,
      compiler_params=pltpu.CompilerParams(
          use_tc_tiling_on_sc=True, needs_layout_passes=False),
      out_type=jax.ShapeDtypeStruct((_NF * _E, _B), jnp.float32),
      scratch_types=[
          pltpu.VMEM((_V,), jnp.float32),
          pltpu.VMEM((_NCHUNK, _CHUNK), jnp.int32),
          pltpu.VMEM((2, _CHUNK), jnp.float32),
          pltpu.SemaphoreType.DMA,
          pltpu.SemaphoreType.DMA,
          pltpu.SemaphoreType.DMA,
          pltpu.SemaphoreType.DMA,
          pltpu.SemaphoreType.DMA,
          pltpu.SemaphoreType.DMA,
          pltpu.SemaphoreType.DMA,
      ],
  )
  def emb_gather(cat_hbm, tab_hbm, out_hbm,
                 row_v, cat_r, out_v, sem_r, sc0, sc1, sc2, sc3, so0, so1):
    sem_c = (sc0, sc1, sc2, sc3)
    sem_o = (so0, so1)
    w = lax.axis_index("s") * 2 + lax.axis_index("c")

    def fire_row(f):
      pltpu.async_copy(tab_hbm.at[f, w], row_v, sem_r)

    def fire_cat(f, k):
      pltpu.async_copy(
          cat_hbm.at[f, pl.ds(k * _CHUNK, _CHUNK)], cat_r.at[k], sem_c[k])

    def wait_cat(f, k):
      pltpu.make_async_copy(
          cat_hbm.at[f, pl.ds(k * _CHUNK, _CHUNK)], cat_r.at[k],
          sem_c[k]).wait()

    def field(f, first, last):
      pltpu.make_async_copy(tab_hbm.at[f, w], row_v, sem_r).wait()
      c = f * _E + w
      for k in range(_NCHUNK):
        buf = k % 2
        dst = out_hbm.at[c, pl.ds(k * _CHUNK, _CHUNK)]
        wait_cat(f, k)
        if k >= 2 or not first:
          pltpu.make_async_copy(out_v.at[buf], dst, sem_o[buf]).wait()

        @plsc.parallel_loop(0, _CHUNK, 16, unroll=_UNROLL)
        def gath(q, _k=k, _buf=buf):
          idx16 = cat_r[_k, pl.ds(q, 16)]
          out_v[_buf, pl.ds(q, 16)] = plsc.load_gather(row_v, [idx16])
        pltpu.async_copy(out_v.at[buf], dst, sem_o[buf])

        if not last:
          if k < 2:
            fire_cat(f, k + 2)
          else:
            fire_cat(f + 1, k - 2)
        elif k < 2:
          fire_cat(f, k + 2)

    fire_row(0)
    fire_cat(0, 0)
    fire_cat(0, 1)
    field(0, first=True, last=False)

    def field_body(f, carry):
      fire_row(f)
      field(f, first=False, last=False)
      return carry

    lax.fori_loop(1, _NF - 1, field_body, 0)

    fire_row(_NF - 1)
    field(_NF - 1, first=False, last=True)

    for buf in range(2):
      k = _NCHUNK - 2 + buf
      dst = out_hbm.at[(_NF - 1) * _E + w, pl.ds(k * _CHUNK, _CHUNK)]
      pltpu.make_async_copy(out_v.at[buf], dst, sem_o[buf]).wait()

  return emb_gather


_EMB_GATHER = _make_kernel()


def kernel(cat_data, tables):
  tab_t = jnp.transpose(tables, (0, 2, 1))
  cat_t = cat_data.astype(jnp.int32).T
  out_t = _EMB_GATHER(cat_t, tab_t)
  return out_t.T


# speedup vs baseline: 1.0045x; 1.0045x over previous
import functools

import jax
import jax.numpy as jnp
from jax import lax
from jax.experimental import pallas as pl
from jax.experimental.pallas import tpu as pltpu
from jax.experimental.pallas import tpu_sc as plsc

_NF = 26
_V = 100000
_E = 32
_B = 16384
_CHUNK = 4096
_NCHUNK = _B // _CHUNK
_UNROLL = 16


def _make_kernel():
  mesh = plsc.VectorSubcoreMesh(core_axis_name="c", subcore_axis_name="s")

  @functools.partial(
      pl.kernel,
      mesh=mesh,
      compiler_params=pltpu.CompilerParams(
          use_tc_tiling_on_sc=True, needs_layout_passes=False),
      out_type=jax.ShapeDtypeStruct((_NF * _E, _B), jnp.float32),
      scratch_types=[
          pltpu.VMEM((_V,), jnp.float32),
          pltpu.VMEM((_NCHUNK, _CHUNK), jnp.int32),
          pltpu.VMEM((2, _CHUNK), jnp.float32),
          pltpu.SemaphoreType.DMA,
          pltpu.SemaphoreType.DMA,
          pltpu.SemaphoreType.DMA,
          pltpu.SemaphoreType.DMA,
          pltpu.SemaphoreType.DMA,
          pltpu.SemaphoreType.DMA,
          pltpu.SemaphoreType.DMA,
      ],
  )
  def emb_gather(cat_hbm, tab_hbm, out_hbm,
                 row_v, cat_r, out_v, sem_r, sc0, sc1, sc2, sc3, so0, so1):
    sem_c = (sc0, sc1, sc2, sc3)
    sem_o = (so0, so1)
    w = lax.axis_index("s") * 2 + lax.axis_index("c")

    def fire_row(f):
      pltpu.async_copy(tab_hbm.at[f, w], row_v, sem_r)

    def fire_cat(f, k):
      pltpu.async_copy(
          cat_hbm.at[f, pl.ds(k * _CHUNK, _CHUNK)], cat_r.at[k], sem_c[k])

    def wait_cat(f, k):
      pltpu.make_async_copy(
          cat_hbm.at[f, pl.ds(k * _CHUNK, _CHUNK)], cat_r.at[k],
          sem_c[k]).wait()

    def field(f, first, last):
      pltpu.make_async_copy(tab_hbm.at[f, w], row_v, sem_r).wait()
      c = f * _E + w
      for k in range(_NCHUNK):
        buf = k % 2
        dst = out_hbm.at[c, pl.ds(k * _CHUNK, _CHUNK)]
        wait_cat(f, k)
        if k >= 2 or not first:
          pltpu.make_async_copy(out_v.at[buf], dst, sem_o[buf]).wait()

        @plsc.parallel_loop(0, _CHUNK, 16, unroll=_UNROLL)
        def gath(q, _k=k, _buf=buf):
          idx16 = cat_r[_k, pl.ds(q, 16)]
          out_v[_buf, pl.ds(q, 16)] = plsc.load_gather(row_v, [idx16])
        pltpu.async_copy(out_v.at[buf], dst, sem_o[buf])

        if not last:
          if k < 2:
            fire_cat(f, k + 2)
          else:
            fire_cat(f + 1, k - 2)
        elif k < 2:
          fire_cat(f, k + 2)

    fire_row(0)
    fire_cat(0, 0)
    fire_cat(0, 1)
    field(0, first=True, last=False)

    def field_body(f, carry):
      fire_row(f)
      field(f, first=False, last=False)
      return carry

    lax.fori_loop(1, _NF - 1, field_body, 0)

    fire_row(_NF - 1)
    field(_NF - 1, first=False, last=True)

    for buf in range(2):
      k = _NCHUNK - 2 + buf
      dst = out_hbm.at[(_NF - 1) * _E + w, pl.ds(k * _CHUNK, _CHUNK)]
      pltpu.make_async_copy(out_v.at[buf], dst, sem_o[buf]).wait()

  return emb_gather


_EMB_GATHER = _make_kernel()


def kernel(cat_data, tables):
  tab_t = jnp.transpose(tables, (0, 2, 1))
  cat_t = cat_data.astype(jnp.int32).T
  out_t = _EMB_GATHER(cat_t, tab_t)
  return out_t.T


# speedup vs baseline: 1.0077x; 1.0032x over previous
import functools

import jax
import jax.numpy as jnp
from jax import lax
from jax.experimental import pallas as pl
from jax.experimental.pallas import tpu as pltpu
from jax.experimental.pallas import tpu_sc as plsc

_NF = 26
_V = 100000
_E = 32
_B = 16384
_CHUNK = 4096
_NCHUNK = _B // _CHUNK
_UNROLL = 8


def _make_kernel():
  mesh = plsc.VectorSubcoreMesh(core_axis_name="c", subcore_axis_name="s")

  @functools.partial(
      pl.kernel,
      mesh=mesh,
      compiler_params=pltpu.CompilerParams(
          use_tc_tiling_on_sc=True, needs_layout_passes=False),
      out_type=jax.ShapeDtypeStruct((_NF * _E, _B), jnp.float32),
      scratch_types=[
          pltpu.VMEM((_V,), jnp.float32),
          pltpu.VMEM((_NCHUNK, _CHUNK), jnp.int32),
          pltpu.VMEM((2, _CHUNK), jnp.float32),
          pltpu.SemaphoreType.DMA,
          pltpu.SemaphoreType.DMA,
          pltpu.SemaphoreType.DMA,
          pltpu.SemaphoreType.DMA,
          pltpu.SemaphoreType.DMA,
          pltpu.SemaphoreType.DMA,
          pltpu.SemaphoreType.DMA,
      ],
  )
  def emb_gather(cat_hbm, tab_hbm, out_hbm,
                 row_v, cat_r, out_v, sem_r, sc0, sc1, sc2, sc3, so0, so1):
    sem_c = (sc0, sc1, sc2, sc3)
    sem_o = (so0, so1)
    w = lax.axis_index("s") * 2 + lax.axis_index("c")

    def fire_row(f):
      pltpu.async_copy(tab_hbm.at[f, w], row_v, sem_r)

    def fire_cat(f, k):
      pltpu.async_copy(
          cat_hbm.at[f, pl.ds(k * _CHUNK, _CHUNK)], cat_r.at[k], sem_c[k])

    def wait_cat(f, k):
      pltpu.make_async_copy(
          cat_hbm.at[f, pl.ds(k * _CHUNK, _CHUNK)], cat_r.at[k],
          sem_c[k]).wait()

    def field(f, first, last):
      pltpu.make_async_copy(tab_hbm.at[f, w], row_v, sem_r).wait()
      c = f * _E + w
      for k in range(_NCHUNK):
        buf = k % 2
        dst = out_hbm.at[c, pl.ds(k * _CHUNK, _CHUNK)]
        wait_cat(f, k)
        if k >= 2 or not first:
          pltpu.make_async_copy(out_v.at[buf], dst, sem_o[buf]).wait()

        @plsc.parallel_loop(0, _CHUNK, 16, unroll=_UNROLL)
        def gath(q, _k=k, _buf=buf):
          idx16 = cat_r[_k, pl.ds(q, 16)]
          out_v[_buf, pl.ds(q, 16)] = plsc.load_gather(row_v, [idx16])
        pltpu.async_copy(out_v.at[buf], dst, sem_o[buf])

        if not last:
          if k < 2:
            fire_cat(f, k + 2)
          else:
            fire_cat(f + 1, k - 2)
        elif k < 2:
          fire_cat(f, k + 2)

    fire_row(0)
    fire_cat(0, 0)
    fire_cat(0, 1)
    field(0, first=True, last=False)

    def field_body(f, carry):
      fire_row(f)
      field(f, first=False, last=False)
      return carry

    lax.fori_loop(1, _NF - 1, field_body, 0)

    fire_row(_NF - 1)
    field(_NF - 1, first=False, last=True)

    for buf in range(2):
      k = _NCHUNK - 2 + buf
      dst = out_hbm.at[(_NF - 1) * _E + w, pl.ds(k * _CHUNK, _CHUNK)]
      pltpu.make_async_copy(out_v.at[buf], dst, sem_o[buf]).wait()

  return emb_gather


_EMB_GATHER = _make_kernel()


def kernel(cat_data, tables):
  tab_t = jnp.transpose(tables, (0, 2, 1))
  cat_t = cat_data.astype(jnp.int32).T
  out_t = _EMB_GATHER(cat_t, tab_t)
  return out_t.T
